# baseline (speedup 1.0000x reference)
import functools

import jax
import jax.numpy as jnp
from jax import lax
from jax.experimental import pallas as pl
from jax.experimental.pallas import tpu as pltpu
from jax.experimental.pallas import tpu_sc as plsc

B = 16384
D = 16
L = 16
NC = 2
NS = 16
NW = NC * NS
CHUNK = B // NW
GCH = 128
NG = CHUNK // GCH


def _sc_margins(uids2d, pids2d, nids2d, user_emb, item_emb):
    mesh = plsc.VectorSubcoreMesh(core_axis_name="c", subcore_axis_name="s")

    @functools.partial(
        pl.kernel,
        out_type=jax.ShapeDtypeStruct((B,), jnp.float32),
        mesh=mesh,
        compiler_params=pltpu.CompilerParams(needs_layout_passes=False, use_tc_tiling_on_sc=False),
        scratch_types=[
            pltpu.VMEM((NG, GCH), jnp.int32),
            pltpu.VMEM((NG, GCH), jnp.int32),
            pltpu.VMEM((NG, GCH), jnp.int32),
            pltpu.VMEM((CHUNK, D), jnp.float32),
            pltpu.VMEM((CHUNK, D), jnp.float32),
            pltpu.VMEM((CHUNK, D), jnp.float32),
            pltpu.VMEM((CHUNK,), jnp.float32),
            pltpu.SemaphoreType.DMA,
        ],
    )
    def body(uids_hbm, pids_hbm, nids_hbm, user_hbm, item_hbm, out_hbm,
             uidx_v, pidx_v, nidx_v, u_v, p_v, n_v, x_v, sem):
        wid = lax.axis_index("s") * NC + lax.axis_index("c")
        row0 = wid * NG

        pltpu.sync_copy(uids_hbm.at[pl.ds(row0, NG)], uidx_v)
        pltpu.sync_copy(pids_hbm.at[pl.ds(row0, NG)], pidx_v)
        pltpu.sync_copy(nids_hbm.at[pl.ds(row0, NG)], nidx_v)

        copies = []
        for k in range(NG):
            dst = pl.ds(k * GCH, GCH)
            copies.append(pltpu.async_copy(user_hbm.at[uidx_v.at[k]], u_v.at[dst], sem))
            copies.append(pltpu.async_copy(item_hbm.at[pidx_v.at[k]], p_v.at[dst], sem))
            copies.append(pltpu.async_copy(item_hbm.at[nidx_v.at[k]], n_v.at[dst], sem))
        for c in copies:
            c.wait()

        def group(i, carry):
            r0 = i * L
            rows = r0 + lax.iota(jnp.int32, L)
            accp = jnp.zeros((L,), jnp.float32)
            accn = jnp.zeros((L,), jnp.float32)
            for j in range(D):
                cols = jnp.full((L,), j, jnp.int32)
                cu = plsc.load_gather(u_v, [rows, cols])
                cp = plsc.load_gather(p_v, [rows, cols])
                cn = plsc.load_gather(n_v, [rows, cols])
                accp = accp + cu * cp
                accn = accn + cu * cn
            x_v[pl.ds(r0, L)] = accp - accn
            return carry

        lax.fori_loop(0, CHUNK // L, group, 0)
        pltpu.sync_copy(x_v, out_hbm.at[pl.ds(wid * CHUNK, CHUNK)])

    return body(uids2d, pids2d, nids2d, user_emb, item_emb)


def _tc_loss(x2d):
    def body(x_ref, o_ref):
        x = x_ref[...]
        p = 1.0 / (1.0 + jnp.exp(-x))
        y = -jnp.log(p + 1e-08)
        o_ref[0, 0] = jnp.sum(y) * (1.0 / B)

    out = pl.pallas_call(
        body,
        out_shape=jax.ShapeDtypeStruct((1, 1), jnp.float32),
        out_specs=pl.BlockSpec(memory_space=pltpu.SMEM),
    )(x2d)
    return out[0, 0]


def kernel(uids, pids, nids, user_emb, item_emb):
    uids2d = uids.astype(jnp.int32).reshape(B // GCH, GCH)
    pids2d = pids.astype(jnp.int32).reshape(B // GCH, GCH)
    nids2d = nids.astype(jnp.int32).reshape(B // GCH, GCH)
    x = _sc_margins(uids2d, pids2d, nids2d, user_emb, item_emb)
    return _tc_loss(x.reshape(B // GCH, GCH))


# speedup vs baseline: 4.8327x; 4.8327x over previous
import functools

import jax
import jax.numpy as jnp
from jax import lax
from jax.experimental import pallas as pl
from jax.experimental.pallas import tpu as pltpu
from jax.experimental.pallas import tpu_sc as plsc

B = 16384
D = 16
N = 1000000
L = 16
NC = 2
TPC = 16
BPT = B // TPC
NV = BPT // L
W = 372 * 128
NWIN = 21
TAIL0 = NWIN * W
NTAIL = N - TAIL0
NB = NWIN + 1
CAP = BPT + NB * L + L
RING = 4


def _sc_rows(uids, pids, nids, uT, iT, utail, itail):
    mesh = plsc.VectorSubcoreMesh(core_axis_name="c", subcore_axis_name="s")
    row_t = jax.ShapeDtypeStruct((B + L, 128), jnp.float32)

    def list_scratch():
        return [
            pltpu.VMEM((BPT,), jnp.int32),
            pltpu.VMEM((CAP,), jnp.int32),
            pltpu.VMEM((CAP,), jnp.int32),
            pltpu.SMEM((NB,), jnp.int32),
            pltpu.SMEM((NB,), jnp.int32),
            pltpu.SMEM((NB,), jnp.int32),
            pltpu.SMEM((NB,), jnp.int32),
        ]

    @functools.partial(
        pl.kernel,
        out_type=(row_t, row_t, row_t),
        mesh=mesh,
        compiler_params=pltpu.CompilerParams(needs_layout_passes=False),
        scratch_types=[
            pltpu.VMEM_SHARED((D * W,), jnp.float32),
            pltpu.VMEM_SHARED((D * W,), jnp.float32),
            pltpu.VMEM((NTAIL * D,), jnp.float32),
            pltpu.VMEM((2, 128), jnp.int32),
            pltpu.VMEM((2, 128), jnp.float32),
            pltpu.VMEM((RING, L, 128), jnp.float32),
            pltpu.SemaphoreType.DMA,
            pltpu.SemaphoreType.DMA,
            pltpu.SemaphoreType.DMA,
        ] + list_scratch() + list_scratch(),
    )
    def body(uids_hbm, pids_hbm, nids_hbm, uT_hbm, iT_hbm, ut_hbm, it_hbm,
             uo_hbm, po_hbm, no_hbm,
             sp0_v, sp1_v, tail_v, idxb_v, asm_v, rows_v, ssem, gsem, csem,
             ids_a, eix_a, pos_a, cnt_a, off_a, cur_a, nch_a,
             ids_b, eix_b, pos_b, cnt_b, off_b, cur_b, nch_b):
        cid = lax.axis_index("c")
        sid = lax.axis_index("s")
        iota = lax.iota(jnp.int32, L)
        dh = iota // 8
        dq = (iota % 8) * L

        def bucketize(ids_hbm_, ids_v, eix_v, pos_v, cnt_s, off_s, cur_s,
                      nch_s):
            base = sid * BPT
            pltpu.sync_copy(ids_hbm_.at[pl.ds(base, BPT)], ids_v)
            for w in range(NB):
                cnt_s[w] = 0

            def cbody(g, c):
                wv = ids_v[pl.ds(g * L, L)] // W
                for w in range(NB):
                    m = wv == w
                    cnt_s[w] = cnt_s[w] + plsc.all_reduce_population_count(m)[0]
                return c

            lax.fori_loop(0, NV, cbody, 0)

            acc = jnp.int32(0)
            for w in range(NB):
                off_s[w] = acc
                cur_s[w] = acc
                nc = (cnt_s[w] + (L - 1)) // L
                nch_s[w] = nc
                acc = acc + nc * L

            def sbody(g, c):
                iv = ids_v[pl.ds(g * L, L)]
                wv = iv // W
                ev = iv - wv * W
                pv = base + g * L + iota
                for w in range(NB):
                    m = wv == w
                    a = cur_s[w]
                    plsc.store_compressed(eix_v.at[pl.ds(a, L)], ev, mask=m)
                    plsc.store_compressed(pos_v.at[pl.ds(a, L)], pv, mask=m)
                    cur_s[w] = a + plsc.all_reduce_population_count(m)[0]
                return c

            lax.fori_loop(0, NV, sbody, 0)
            zv = jnp.zeros((L,), jnp.int32)
            dumpv = B + iota
            for w in range(NB):
                a = cur_s[w]
                npad = off_s[w] + nch_s[w] * L - a
                m = iota < npad
                plsc.store_compressed(eix_v.at[pl.ds(a, L)], zv, mask=m)
                plsc.store_compressed(pos_v.at[pl.ds(a, L)], dumpv, mask=m)

        def stage(tab_hbm, w, sp_buf):
            c0 = w * W
            return [
                pltpu.async_copy(tab_hbm.at[r, pl.ds(c0, W)],
                                 sp_buf.at[pl.ds(r * W, W)], ssem)
                for r in range(D)
            ]

        def drain_stage(tab_hbm, sp_buf):
            for r in range(D):
                pltpu.make_async_copy(tab_hbm.at[r, pl.ds(0, W)],
                                      sp_buf.at[pl.ds(r * W, W)],
                                      ssem).wait()

        def emit_chunk(out_hbm, eix_v, pos_v, off, c, gather_vals):
            ev = eix_v[pl.ds(off, L)]
            gather_vals(ev)
            rb = c % RING
            for l in range(D):
                rows_v[rb, l, pl.ds(0, L)] = plsc.load_gather(asm_v, [dh, dq + l])
            pv = pos_v[pl.ds(off, L)]
            pltpu.async_copy(rows_v.at[rb], out_hbm.at[pv], csem).wait()

        def drain_scatter1(out_hbm):
            pltpu.make_async_copy(rows_v.at[0],
                                  out_hbm.at[pl.ds(0, L)], csem).wait()

        def spmem_gather(sp_buf):
            def g(ev):
                for d in range(D):
                    idxb_v[d // 8, pl.ds((d % 8) * L, L)] = ev + d * W
                c1 = pltpu.async_copy(sp_buf.at[idxb_v.at[0]],
                                      asm_v.at[0], gsem)
                c2 = pltpu.async_copy(sp_buf.at[idxb_v.at[1]],
                                      asm_v.at[1], gsem)
                c1.wait()
                c2.wait()
            return g

        def tail_gather(ev):
            evd = ev * D
            for d in range(D):
                vals = plsc.load_gather(tail_v, [evd + d])
                asm_v[d // 8, pl.ds((d % 8) * L, L)] = vals

        def run_core(tab_hbm, tail_hbm, lists):
            pltpu.sync_copy(tail_hbm, tail_v)

            def half(w, sp_buf, sp_other, fire_next=True):
                @pl.when(sid == 0)
                def _():
                    drain_stage(tab_hbm, sp_buf)

                plsc.subcore_barrier()

                if fire_next:
                    @pl.when(jnp.logical_and(sid == 0, w + 1 < NWIN))
                    def _():
                        stage(tab_hbm, w + 1, sp_other)

                for (eix_v, pos_v, off_s, nch_s, out_hbm) in lists:
                    def chbody(ch, c2, eix_v=eix_v, pos_v=pos_v,
                               off_s=off_s, out_hbm=out_hbm):
                        emit_chunk(out_hbm, eix_v, pos_v,
                                   off_s[w] + ch * L, ch,
                                   spmem_gather(sp_buf))
                        return c2

                    lax.fori_loop(0, nch_s[w], chbody, 0)
                plsc.subcore_barrier()

            def wbody(k, c):
                half(2 * k, sp0_v, sp1_v)
                half(2 * k + 1, sp1_v, sp0_v)
                return c

            lax.fori_loop(0, NWIN // 2, wbody, 0)
            if NWIN % 2:
                half(NWIN - 1, sp0_v, sp1_v, fire_next=False)

            for (eix_v, pos_v, off_s, nch_s, out_hbm) in lists:
                def tbody(ch, c2, eix_v=eix_v, pos_v=pos_v, off_s=off_s,
                          out_hbm=out_hbm):
                    emit_chunk(out_hbm, eix_v, pos_v,
                               off_s[NWIN] + ch * L, ch, tail_gather)
                    return c2

                lax.fori_loop(0, nch_s[NWIN], tbody, 0)

        @pl.when(cid == 0)
        def _():
            @pl.when(sid == 0)
            def _():
                stage(uT_hbm, 0, sp0_v)

            bucketize(uids_hbm, ids_a, eix_a, pos_a, cnt_a, off_a, cur_a,
                      nch_a)
            run_core(uT_hbm, ut_hbm,
                     [(eix_a, pos_a, off_a, nch_a, uo_hbm)])

        @pl.when(cid == 1)
        def _():
            @pl.when(sid == 0)
            def _():
                stage(iT_hbm, 0, sp0_v)

            bucketize(pids_hbm, ids_a, eix_a, pos_a, cnt_a, off_a, cur_a,
                      nch_a)
            bucketize(nids_hbm, ids_b, eix_b, pos_b, cnt_b, off_b, cur_b,
                      nch_b)
            run_core(iT_hbm, it_hbm,
                     [(eix_a, pos_a, off_a, nch_a, po_hbm),
                      (eix_b, pos_b, off_b, nch_b, no_hbm)])

    return body(uids, pids, nids, uT, iT, utail, itail)


def _tc_loss(u, p, n):
    def tcbody(u_ref, p_ref, n_ref, o_ref):
        uu = u_ref[pl.ds(0, B), pl.ds(0, D)]
        pp = p_ref[pl.ds(0, B), pl.ds(0, D)]
        nn = n_ref[pl.ds(0, B), pl.ds(0, D)]
        x = jnp.sum(uu * (pp - nn), axis=1)
        sg = 1.0 / (1.0 + jnp.exp(-x))
        y = -jnp.log(sg + 1e-08)
        o_ref[0, 0] = jnp.sum(y) * (1.0 / B)

    out = pl.pallas_call(
        tcbody,
        out_shape=jax.ShapeDtypeStruct((1, 1), jnp.float32),
        out_specs=pl.BlockSpec(memory_space=pltpu.SMEM),
    )(u, p, n)
    return out[0, 0]


def kernel(uids, pids, nids, user_emb, item_emb):
    uids = uids.astype(jnp.int32)
    pids = pids.astype(jnp.int32)
    nids = nids.astype(jnp.int32)
    urows, prows, nrows = _sc_rows(
        uids, pids, nids, user_emb.T, item_emb.T,
        user_emb[TAIL0:].reshape(-1), item_emb[TAIL0:].reshape(-1))
    return _tc_loss(urows, prows, nrows)


# speedup vs baseline: 5.0742x; 1.0500x over previous
import functools

import jax
import jax.numpy as jnp
from jax import lax
from jax.experimental import pallas as pl
from jax.experimental.pallas import tpu as pltpu
from jax.experimental.pallas import tpu_sc as plsc

B = 16384
D = 16
N = 1000000
L = 16
NC = 2
TPC = 16
BPT = B // TPC
NV = BPT // L
W = 372 * 128
NWIN = 21
TAIL0 = NWIN * W
NTAIL = N - TAIL0
NB = NWIN + 1
CAP = BPT + NB * L + L
RING = 4


def _sc_rows(uids, pids, nids, uT, iT, utail, itail):
    mesh = plsc.VectorSubcoreMesh(core_axis_name="c", subcore_axis_name="s")
    row_t = jax.ShapeDtypeStruct((B + L, 128), jnp.float32)

    def list_scratch():
        return [
            pltpu.VMEM((BPT,), jnp.int32),
            pltpu.VMEM((CAP,), jnp.int32),
            pltpu.VMEM((CAP,), jnp.int32),
            pltpu.SMEM((NB,), jnp.int32),
            pltpu.SMEM((NB,), jnp.int32),
            pltpu.SMEM((NB,), jnp.int32),
            pltpu.SMEM((NB,), jnp.int32),
        ]

    @functools.partial(
        pl.kernel,
        out_type=(row_t, row_t, row_t),
        mesh=mesh,
        compiler_params=pltpu.CompilerParams(needs_layout_passes=False),
        scratch_types=[
            pltpu.VMEM_SHARED((D * W,), jnp.float32),
            pltpu.VMEM_SHARED((D * W,), jnp.float32),
            pltpu.VMEM((NTAIL * D,), jnp.float32),
            pltpu.VMEM((2, 128), jnp.int32),
            pltpu.VMEM((2, 128), jnp.float32),
            pltpu.VMEM((RING, L, 128), jnp.float32),
            pltpu.SemaphoreType.DMA,
            pltpu.SemaphoreType.DMA,
            pltpu.SemaphoreType.DMA,
        ] + list_scratch() + list_scratch(),
    )
    def body(uids_hbm, pids_hbm, nids_hbm, uT_hbm, iT_hbm, ut_hbm, it_hbm,
             uo_hbm, po_hbm, no_hbm,
             sp0_v, sp1_v, tail_v, idxb_v, asm_v, rows_v, ssem, gsem, csem,
             ids_a, eix_a, pos_a, cnt_a, off_a, cur_a, nch_a,
             ids_b, eix_b, pos_b, cnt_b, off_b, cur_b, nch_b):
        cid = lax.axis_index("c")
        sid = lax.axis_index("s")
        iota = lax.iota(jnp.int32, L)
        dh = iota // 8
        dq = (iota % 8) * L

        def bucketize(ids_hbm_, ids_v, eix_v, pos_v, cnt_s, off_s, cur_s,
                      nch_s):
            base = sid * BPT
            pltpu.sync_copy(ids_hbm_.at[pl.ds(base, BPT)], ids_v)
            for w in range(NB):
                cnt_s[w] = 0

            def cbody(g, c):
                wv = ids_v[pl.ds(g * L, L)] // W
                for w in range(NB):
                    m = wv == w
                    cnt_s[w] = cnt_s[w] + plsc.all_reduce_population_count(m)[0]
                return c

            lax.fori_loop(0, NV, cbody, 0)

            acc = jnp.int32(0)
            for w in range(NB):
                off_s[w] = acc
                cur_s[w] = acc
                nc = (cnt_s[w] + (L - 1)) // L
                nch_s[w] = nc
                acc = acc + nc * L

            def sbody(g, c):
                iv = ids_v[pl.ds(g * L, L)]
                wv = iv // W
                ev = iv - wv * W
                pv = base + g * L + iota
                for w in range(NB):
                    m = wv == w
                    a = cur_s[w]
                    plsc.store_compressed(eix_v.at[pl.ds(a, L)], ev, mask=m)
                    plsc.store_compressed(pos_v.at[pl.ds(a, L)], pv, mask=m)
                    cur_s[w] = a + plsc.all_reduce_population_count(m)[0]
                return c

            lax.fori_loop(0, NV, sbody, 0)
            zv = jnp.zeros((L,), jnp.int32)
            dumpv = B + iota
            for w in range(NB):
                a = cur_s[w]
                npad = off_s[w] + nch_s[w] * L - a
                m = iota < npad
                plsc.store_compressed(eix_v.at[pl.ds(a, L)], zv, mask=m)
                plsc.store_compressed(pos_v.at[pl.ds(a, L)], dumpv, mask=m)

        def stage(tab_hbm, w, sp_buf):
            c0 = w * W
            pltpu.async_copy(tab_hbm.at[sid, pl.ds(c0, W)],
                             sp_buf.at[pl.ds(sid * W, W)], ssem)

        def drain_stage(tab_hbm, sp_buf):
            pltpu.make_async_copy(tab_hbm.at[sid, pl.ds(0, W)],
                                  sp_buf.at[pl.ds(sid * W, W)],
                                  ssem).wait()

        def emit_chunk(out_hbm, eix_v, pos_v, off, c, gather_vals):
            ev = eix_v[pl.ds(off, L)]
            gather_vals(ev)
            rb = c % RING
            for l in range(D):
                rows_v[rb, l, pl.ds(0, L)] = plsc.load_gather(asm_v, [dh, dq + l])
            pv = pos_v[pl.ds(off, L)]
            pltpu.async_copy(rows_v.at[rb], out_hbm.at[pv], csem)

        def drain_scatter1(out_hbm):
            pltpu.make_async_copy(rows_v.at[0],
                                  out_hbm.at[pl.ds(0, L)], csem).wait()

        def spmem_gather(sp_buf):
            def g(ev):
                for d in range(D):
                    idxb_v[d // 8, pl.ds((d % 8) * L, L)] = ev + d * W
                c1 = pltpu.async_copy(sp_buf.at[idxb_v.at[0]],
                                      asm_v.at[0], gsem)
                c2 = pltpu.async_copy(sp_buf.at[idxb_v.at[1]],
                                      asm_v.at[1], gsem)
                c1.wait()
                c2.wait()
            return g

        def tail_gather(ev):
            evd = ev * D
            for d in range(D):
                vals = plsc.load_gather(tail_v, [evd + d])
                asm_v[d // 8, pl.ds((d % 8) * L, L)] = vals

        def run_core(tab_hbm, tail_hbm, lists):
            pltpu.sync_copy(tail_hbm, tail_v)

            def half(w, sp_buf, sp_other, fire_next=True):
                drain_stage(tab_hbm, sp_buf)
                plsc.subcore_barrier()

                if fire_next:
                    @pl.when(w + 1 < NWIN)
                    def _():
                        stage(tab_hbm, w + 1, sp_other)

                for (eix_v, pos_v, off_s, nch_s, out_hbm) in lists:
                    def chbody(ch, c2, eix_v=eix_v, pos_v=pos_v,
                               off_s=off_s, out_hbm=out_hbm):
                        emit_chunk(out_hbm, eix_v, pos_v,
                                   off_s[w] + ch * L, ch,
                                   spmem_gather(sp_buf))

                        @pl.when(ch >= RING)
                        def _():
                            drain_scatter1(out_hbm)

                        return c2

                    nw = nch_s[w]
                    lax.fori_loop(0, nw, chbody, 0)

                    def dbody(ch, c2, out_hbm=out_hbm):
                        drain_scatter1(out_hbm)
                        return c2

                    lax.fori_loop(0, jnp.minimum(nw, RING), dbody, 0)
                plsc.subcore_barrier()

            def wbody(k, c):
                half(2 * k, sp0_v, sp1_v)
                half(2 * k + 1, sp1_v, sp0_v)
                return c

            lax.fori_loop(0, NWIN // 2, wbody, 0)
            if NWIN % 2:
                half(NWIN - 1, sp0_v, sp1_v, fire_next=False)

            for (eix_v, pos_v, off_s, nch_s, out_hbm) in lists:
                def tbody(ch, c2, eix_v=eix_v, pos_v=pos_v, off_s=off_s,
                          out_hbm=out_hbm):
                    emit_chunk(out_hbm, eix_v, pos_v,
                               off_s[NWIN] + ch * L, ch, tail_gather)

                    @pl.when(ch >= RING)
                    def _():
                        drain_scatter1(out_hbm)

                    return c2

                nt = nch_s[NWIN]
                lax.fori_loop(0, nt, tbody, 0)

                def dtbody(ch, c2, out_hbm=out_hbm):
                    drain_scatter1(out_hbm)
                    return c2

                lax.fori_loop(0, jnp.minimum(nt, RING), dtbody, 0)

        @pl.when(cid == 0)
        def _():
            stage(uT_hbm, 0, sp0_v)
            bucketize(uids_hbm, ids_a, eix_a, pos_a, cnt_a, off_a, cur_a,
                      nch_a)
            run_core(uT_hbm, ut_hbm,
                     [(eix_a, pos_a, off_a, nch_a, uo_hbm)])

        @pl.when(cid == 1)
        def _():
            stage(iT_hbm, 0, sp0_v)
            bucketize(pids_hbm, ids_a, eix_a, pos_a, cnt_a, off_a, cur_a,
                      nch_a)
            bucketize(nids_hbm, ids_b, eix_b, pos_b, cnt_b, off_b, cur_b,
                      nch_b)
            run_core(iT_hbm, it_hbm,
                     [(eix_a, pos_a, off_a, nch_a, po_hbm),
                      (eix_b, pos_b, off_b, nch_b, no_hbm)])

    return body(uids, pids, nids, uT, iT, utail, itail)


def _tc_loss(u, p, n):
    def tcbody(u_ref, p_ref, n_ref, o_ref):
        uu = u_ref[pl.ds(0, B), pl.ds(0, D)]
        pp = p_ref[pl.ds(0, B), pl.ds(0, D)]
        nn = n_ref[pl.ds(0, B), pl.ds(0, D)]
        x = jnp.sum(uu * (pp - nn), axis=1)
        sg = 1.0 / (1.0 + jnp.exp(-x))
        y = -jnp.log(sg + 1e-08)
        o_ref[0, 0] = jnp.sum(y) * (1.0 / B)

    out = pl.pallas_call(
        tcbody,
        out_shape=jax.ShapeDtypeStruct((1, 1), jnp.float32),
        out_specs=pl.BlockSpec(memory_space=pltpu.SMEM),
    )(u, p, n)
    return out[0, 0]


def kernel(uids, pids, nids, user_emb, item_emb):
    uids = uids.astype(jnp.int32)
    pids = pids.astype(jnp.int32)
    nids = nids.astype(jnp.int32)
    urows, prows, nrows = _sc_rows(
        uids, pids, nids, user_emb.T, item_emb.T,
        user_emb[TAIL0:].reshape(-1), item_emb[TAIL0:].reshape(-1))
    return _tc_loss(urows, prows, nrows)
